# baseline (speedup 1.0000x reference)
import jax
import jax.numpy as jnp
from jax import lax
from jax.experimental import pallas as pl
from jax.experimental.pallas import tpu as pltpu
from jax.experimental.pallas import tpu_sc as plsc

VOCAB = 30522
HID = 768
PROMPT = 20
B = 32
S = 512
EPS = 1e-12
L = 16
NH = HID // L
NW = 32
SW = S // NW


def _lane_perm(x, idx):
    dn = lax.GatherDimensionNumbers(offset_dims=(), collapsed_slice_dims=(0,),
                                    start_index_map=(0,))
    return lax.gather(x, idx[:, None], dn, slice_sizes=(1,),
                      mode=lax.GatherScatterMode.PROMISE_IN_BOUNDS)


def _allsum(v):
    idx = lax.iota(jnp.int32, L)
    for sft in (8, 4, 2, 1):
        v = v + _lane_perm(v, jnp.bitwise_xor(idx, sft))
    return v


def _rsqrt_vec(v):
    i = lax.bitcast_convert_type(v, jnp.int32)
    i = jnp.int32(0x5F3759DF) - lax.shift_right_arithmetic(
        i, jnp.full((L,), 1, jnp.int32))
    y = lax.bitcast_convert_type(i, jnp.float32)
    for _ in range(3):
        y = y * (1.5 - 0.5 * v * y * y)
    return y


def _sc_body(ids_hbm, word_hbm, pos_hbm, type_hbm, prompt_hbm, gamma_hbm,
             beta_hbm, out_hbm, idx_v, rbuf0, rbuf1, obuf0, obuf1, pos_v,
             prompt_v, type_v, gamma_v, beta_v, gsem0, gsem1, wsem0, wsem1):
    rbuf = (rbuf0, rbuf1)
    obuf = (obuf0, obuf1)
    gsem = (gsem0, gsem1)
    wsem = (wsem0, wsem1)
    cid = lax.axis_index("c")
    sid = lax.axis_index("s")
    wid = sid * 2 + cid
    s0 = wid * SW

    pltpu.sync_copy(ids_hbm.at[pl.ds(wid * (B * SW), B * SW)], idx_v)
    pltpu.sync_copy(pos_hbm.at[pl.ds(s0, SW)], pos_v)
    pltpu.sync_copy(type_hbm.at[0], type_v)
    pltpu.sync_copy(gamma_hbm, gamma_v)
    pltpu.sync_copy(beta_hbm, beta_v)

    has_prompt = wid <= 1

    @pl.when(has_prompt)
    def _():
        pltpu.sync_copy(prompt_hbm.at[jnp.minimum(wid, 1)], prompt_v)

    def _addtype(t, c):
        for j in range(NH):
            sl = pl.ds(j * L, L)
            pos_v[t, sl] = pos_v[t, sl] + type_v[sl]
        return c
    lax.fori_loop(0, SW, _addtype, 0)

    def _make_token(use_prompt, rows_v, out_v):
        def _token(t, c2):
            if use_prompt:
                s = s0 + t
                inp = jnp.logical_and(s >= 1, s < 1 + PROMPT)
                pm = jnp.full((L,), jnp.where(inp, 1.0, 0.0), jnp.float32)
            acc = jnp.zeros((L,), jnp.float32)
            acc2 = jnp.zeros((L,), jnp.float32)
            for j in range(NH):
                sl = pl.ds(j * L, L)
                xr = rows_v[t, sl]
                if use_prompt:
                    xr = xr + pm * (prompt_v[t, sl] - xr)
                x = xr + pos_v[t, sl]
                out_v[t, sl] = x
                acc = acc + x
                acc2 = acc2 + x * x
            meanv = _allsum(acc) * (1.0 / HID)
            varv = _allsum(acc2) * (1.0 / HID) - meanv * meanv
            rstd = _rsqrt_vec(varv + EPS)
            for j in range(NH):
                sl = pl.ds(j * L, L)
                out_v[t, sl] = ((out_v[t, sl] - meanv) * rstd
                                * gamma_v[sl] + beta_v[sl])
            return c2
        return _token

    def _gather(b, k):
        return pltpu.make_async_copy(
            word_hbm.at[idx_v.at[pl.ds(b * SW, SW)]], rbuf[k], gsem[k])

    for k in (0, 1):
        _gather(k, k).start()

    def _pair(g, c):
        for k in (0, 1):
            b = g * 2 + k
            _gather(b, k).wait()

            @pl.when(b >= 2)
            def _():
                pltpu.make_async_copy(
                    obuf[k], out_hbm.at[b - 2, pl.ds(s0, SW)],
                    wsem[k]).wait()

            @pl.when(has_prompt)
            def _():
                lax.fori_loop(0, SW, _make_token(True, rbuf[k], obuf[k]), 0)

            @pl.when(jnp.logical_not(has_prompt))
            def _():
                lax.fori_loop(0, SW, _make_token(False, rbuf[k], obuf[k]), 0)

            @pl.when(b + 2 < B)
            def _():
                _gather(b + 2, k).start()

            pltpu.make_async_copy(
                obuf[k], out_hbm.at[b, pl.ds(s0, SW)], wsem[k]).start()
        return c
    lax.fori_loop(0, B // 2, _pair, 0)

    for k in (0, 1):
        pltpu.make_async_copy(
            obuf[k], out_hbm.at[B - 2 + k, pl.ds(s0, SW)], wsem[k]).wait()


def kernel(input_ids, word_emb, pos_emb, type_emb, prompt_emb, gamma, beta):
    ids_flat = (input_ids.reshape(B, NW, SW).transpose(1, 0, 2)
                .reshape(NW * B * SW))
    t = jnp.arange(SW)
    prompt_tiles = jnp.stack([
        prompt_emb[jnp.clip(t - 1, 0, PROMPT - 1)],
        prompt_emb[jnp.clip(t + SW - 1, 0, PROMPT - 1)],
    ])

    mesh = plsc.VectorSubcoreMesh(core_axis_name="c", subcore_axis_name="s")
    f = pl.kernel(
        _sc_body,
        out_type=jax.ShapeDtypeStruct((B, S, HID), jnp.float32),
        mesh=mesh,
        scratch_types=[
            pltpu.VMEM((B * SW,), jnp.int32),
            pltpu.VMEM((SW, HID), jnp.float32),
            pltpu.VMEM((SW, HID), jnp.float32),
            pltpu.VMEM((SW, HID), jnp.float32),
            pltpu.VMEM((SW, HID), jnp.float32),
            pltpu.VMEM((SW, HID), jnp.float32),
            pltpu.VMEM((SW, HID), jnp.float32),
            pltpu.VMEM((HID,), jnp.float32),
            pltpu.VMEM((HID,), jnp.float32),
            pltpu.VMEM((HID,), jnp.float32),
            pltpu.SemaphoreType.DMA,
            pltpu.SemaphoreType.DMA,
            pltpu.SemaphoreType.DMA,
            pltpu.SemaphoreType.DMA,
        ],
    )
    return f(ids_flat, word_emb, pos_emb, type_emb, prompt_tiles, gamma, beta)


# speedup vs baseline: 1.1686x; 1.1686x over previous
import jax
import jax.numpy as jnp
from jax import lax
from jax.experimental import pallas as pl
from jax.experimental.pallas import tpu as pltpu
from jax.experimental.pallas import tpu_sc as plsc

VOCAB = 30522
HID = 768
PROMPT = 20
B = 32
S = 512
EPS = 1e-12
L = 16
NH = HID // L
NW = 32
SW = S // NW
SROW = 2 * L + 1


def _allsum(v):
    idx = lax.iota(jnp.int32, L)
    for sft in (8, 4, 2, 1):
        v = v + _lane_perm(v, jnp.bitwise_xor(idx, sft))
    return v


def _lane_perm(x, idx):
    dn = lax.GatherDimensionNumbers(offset_dims=(), collapsed_slice_dims=(0,),
                                    start_index_map=(0,))
    return lax.gather(x, idx[:, None], dn, slice_sizes=(1,),
                      mode=lax.GatherScatterMode.PROMISE_IN_BOUNDS)


def _rsqrt_vec(v):
    i = lax.bitcast_convert_type(v, jnp.int32)
    i = jnp.int32(0x5F3759DF) - lax.shift_right_arithmetic(
        i, jnp.full((L,), 1, jnp.int32))
    y = lax.bitcast_convert_type(i, jnp.float32)
    for _ in range(3):
        y = y * (1.5 - 0.5 * v * y * y)
    return y


def _sc_body(ids_hbm, word_hbm, pos_hbm, type_hbm, prompt_hbm, gamma_hbm,
             beta_hbm, out_hbm, idx_v, rbuf0, rbuf1, obuf0, obuf1, pos_v,
             prompt_v, type_v, gamma_v, beta_v,
             gsem0, gsem1, wsem0, wsem1):
    rbuf = (rbuf0, rbuf1)
    obuf = (obuf0, obuf1)
    gsem = (gsem0, gsem1)
    wsem = (wsem0, wsem1)

    cid = lax.axis_index("c")
    sid = lax.axis_index("s")
    wid = sid * 2 + cid
    s0 = wid * SW

    pltpu.sync_copy(ids_hbm.at[pl.ds(wid * (B * SW), B * SW)], idx_v)
    pltpu.sync_copy(pos_hbm.at[pl.ds(s0, SW)], pos_v)
    pltpu.sync_copy(type_hbm.at[0], type_v)
    pltpu.sync_copy(gamma_hbm, gamma_v)
    pltpu.sync_copy(beta_hbm, beta_v)
    pltpu.sync_copy(prompt_hbm.at[jnp.minimum(wid, 1)], prompt_v)

    p_lo = jnp.clip(1 - s0, 0, SW)
    p_hi = jnp.clip(1 + PROMPT - s0, 0, SW)

    def _addtype(t, c):
        for j in range(NH):
            sl = pl.ds(j * L, L)
            pos_v[t, sl] = pos_v[t, sl] + type_v[sl]
        return c
    lax.fori_loop(0, SW, _addtype, 0)

    def _make_token(src, out_v):
        def _token(t, c):
            a = [jnp.zeros((L,), jnp.float32) for _ in range(4)]
            q = [jnp.zeros((L,), jnp.float32) for _ in range(4)]
            for j in range(NH):
                sl = pl.ds(j * L, L)
                x = src[t, sl] + pos_v[t, sl]
                out_v[t, sl] = x
                a[j & 3] = a[j & 3] + x
                q[j & 3] = q[j & 3] + x * x
            meanv = _allsum((a[0] + a[1]) + (a[2] + a[3])) * (1.0 / HID)
            varv = (_allsum((q[0] + q[1]) + (q[2] + q[3])) * (1.0 / HID)
                    - meanv * meanv)
            rstd = _rsqrt_vec(varv + EPS)
            for j in range(NH):
                sl = pl.ds(j * L, L)
                out_v[t, sl] = ((out_v[t, sl] - meanv) * rstd
                                * gamma_v[sl] + beta_v[sl])
            return c
        return _token

    def _compute(rb, ob):
        lax.fori_loop(0, p_lo, _make_token(rb, ob), 0)
        lax.fori_loop(p_lo, p_hi, _make_token(prompt_v, ob), 0)
        lax.fori_loop(p_hi, SW, _make_token(rb, ob), 0)

    def _gather(b, k):
        return pltpu.make_async_copy(
            word_hbm.at[idx_v.at[pl.ds(b * SW, SW)]], rbuf[k], gsem[k])

    for k in (0, 1):
        _gather(k, k).start()

    def _pair(g, c):
        for k in (0, 1):
            b = g * 2 + k
            _gather(b, k).wait()

            @pl.when(b >= 2)
            def _():
                pltpu.make_async_copy(
                    obuf[k], out_hbm.at[b - 2, pl.ds(s0, SW)],
                    wsem[k]).wait()

            _compute(rbuf[k], obuf[k])

            @pl.when(b + 2 < B)
            def _():
                _gather(b + 2, k).start()

            pltpu.make_async_copy(
                obuf[k], out_hbm.at[b, pl.ds(s0, SW)], wsem[k]).start()
        return c
    lax.fori_loop(0, B // 2, _pair, 0)

    for k in (0, 1):
        pltpu.make_async_copy(
            obuf[k], out_hbm.at[B - 2 + k, pl.ds(s0, SW)], wsem[k]).wait()


def kernel(input_ids, word_emb, pos_emb, type_emb, prompt_emb, gamma, beta):
    ids_flat = (input_ids.reshape(B, NW, SW).transpose(1, 0, 2)
                .reshape(NW * B * SW))
    t = jnp.arange(SW)
    prompt_tiles = jnp.stack([
        prompt_emb[jnp.clip(t - 1, 0, PROMPT - 1)],
        prompt_emb[jnp.clip(t + SW - 1, 0, PROMPT - 1)],
    ])

    mesh = plsc.VectorSubcoreMesh(core_axis_name="c", subcore_axis_name="s")
    f = pl.kernel(
        _sc_body,
        out_type=jax.ShapeDtypeStruct((B, S, HID), jnp.float32),
        mesh=mesh,
        scratch_types=[
            pltpu.VMEM((B * SW,), jnp.int32),
            pltpu.VMEM((SW, HID), jnp.float32),
            pltpu.VMEM((SW, HID), jnp.float32),
            pltpu.VMEM((SW, HID), jnp.float32),
            pltpu.VMEM((SW, HID), jnp.float32),
            pltpu.VMEM((SW, HID), jnp.float32),
            pltpu.VMEM((SW, HID), jnp.float32),
            pltpu.VMEM((HID,), jnp.float32),
            pltpu.VMEM((HID,), jnp.float32),
            pltpu.VMEM((HID,), jnp.float32),
            pltpu.SemaphoreType.DMA,
            pltpu.SemaphoreType.DMA,
            pltpu.SemaphoreType.DMA,
            pltpu.SemaphoreType.DMA,
        ],
    )
    return f(ids_flat, word_emb, pos_emb, type_emb, prompt_tiles, gamma, beta)


# speedup vs baseline: 2.8600x; 2.4473x over previous
import functools

import jax
import jax.numpy as jnp
from jax import lax
from jax.experimental import pallas as pl
from jax.experimental.pallas import tpu as pltpu
from jax.experimental.pallas import tpu_sc as plsc

VOCAB = 30522
HID = 768
PROMPT = 20
B = 32
S = 512
EPS = 1e-12
NW = 32
SW = S // NW
NBUF = 4


def _sc_gather_body(ids_hbm, word_hbm, out_hbm, idx_v, b0, b1, b2, b3,
                    g0, g1, g2, g3, w0, w1, w2, w3):
    buf = (b0, b1, b2, b3)
    gsem = (g0, g1, g2, g3)
    wsem = (w0, w1, w2, w3)

    cid = lax.axis_index("c")
    sid = lax.axis_index("s")
    wid = sid * 2 + cid
    s0 = wid * SW

    pltpu.sync_copy(ids_hbm.at[pl.ds(wid * (B * SW), B * SW)], idx_v)

    def _gather(b, k):
        return pltpu.make_async_copy(
            word_hbm.at[idx_v.at[pl.ds(b * SW, SW)]], buf[k], gsem[k])

    def _write(b, k):
        return pltpu.make_async_copy(
            buf[k], out_hbm.at[b, pl.ds(s0, SW)], wsem[k])

    _gather(0, 0).start()
    _gather(1, 1).start()

    def _quad(g, c):
        for k in range(NBUF):
            b = g * NBUF + k
            _gather(b, k).wait()
            _write(b, k).start()

            kk = (k + 2) % NBUF

            @pl.when(b + 2 < B)
            def _():
                @pl.when(b >= 2)
                def _():
                    _write(b - 2, kk).wait()
                _gather(b + 2, kk).start()
        return c
    lax.fori_loop(0, B // NBUF, _quad, 0)

    for b in range(B - NBUF, B):
        _write(b, b % NBUF).wait()


def _tc_ln_body(inter_ref, pos_ref, type_ref, prompt_ref, gamma_ref,
                beta_ref, out_ref):
    x = inter_ref[0]
    x = jnp.concatenate(
        [x[0:1], prompt_ref[0:PROMPT], x[1 + PROMPT:]], axis=0)
    x = x + pos_ref[...] + type_ref[0][None, :]
    mean = jnp.mean(x, axis=-1, keepdims=True)
    xc = x - mean
    var = jnp.mean(xc * xc, axis=-1, keepdims=True)
    y = xc * lax.rsqrt(var + EPS)
    out_ref[0] = y * gamma_ref[...][None, :] + beta_ref[...][None, :]


def kernel(input_ids, word_emb, pos_emb, type_emb, prompt_emb, gamma, beta):
    ids_flat = (input_ids.reshape(B, NW, SW).transpose(1, 0, 2)
                .reshape(NW * B * SW))

    mesh = plsc.VectorSubcoreMesh(core_axis_name="c", subcore_axis_name="s")
    gathered = pl.kernel(
        _sc_gather_body,
        out_type=jax.ShapeDtypeStruct((B, S, HID), jnp.float32),
        mesh=mesh,
        scratch_types=(
            [pltpu.VMEM((B * SW,), jnp.int32)]
            + [pltpu.VMEM((SW, HID), jnp.float32)] * NBUF
            + [pltpu.SemaphoreType.DMA] * (2 * NBUF)
        ),
    )(ids_flat, word_emb)

    prompt_pad = jnp.pad(prompt_emb, ((0, 4), (0, 0)))
    type_pad = jnp.pad(type_emb, ((0, 6), (0, 0)))

    return pl.pallas_call(
        _tc_ln_body,
        out_shape=jax.ShapeDtypeStruct((B, S, HID), jnp.float32),
        grid=(B,),
        in_specs=[
            pl.BlockSpec((1, S, HID), lambda b: (b, 0, 0)),
            pl.BlockSpec((S, HID), lambda b: (0, 0)),
            pl.BlockSpec((8, HID), lambda b: (0, 0)),
            pl.BlockSpec((PROMPT + 4, HID), lambda b: (0, 0)),
            pl.BlockSpec((HID,), lambda b: (0,)),
            pl.BlockSpec((HID,), lambda b: (0,)),
        ],
        out_specs=pl.BlockSpec((1, S, HID), lambda b: (b, 0, 0)),
    )(gathered, pos_emb, type_pad, prompt_pad, gamma, beta)
